# Initial kernel scaffold; baseline (speedup 1.0000x reference)
import jax
import jax.numpy as jnp
from jax.experimental import pallas as pl


def kernel(x, edge_index, W1, b1, W2, b2, W3, b3):
    raise NotImplementedError("write your pallas kernel here")


# speedup vs baseline: 12.2376x; 12.2376x over previous
import functools

import jax
import jax.numpy as jnp
from jax import lax
from jax.experimental import pallas as pl
from jax.experimental.pallas import tpu as pltpu
from jax.experimental.pallas import tpu_sc as plsc

N = 10000
NPAD = 10240
E = 320000
NC, NS, L = 2, 16, 16
NW = NC * NS
EPT = E // NW
K = 80
NCHUNK = EPT // K
RPT = NPAD // NS

_mesh = plsc.VectorSubcoreMesh(core_axis_name="c", subcore_axis_name="s")
_sc_params = pltpu.CompilerParams(needs_layout_passes=False,
                                  use_tc_tiling_on_sc=False)


@functools.partial(
    pl.kernel,
    out_type=jax.ShapeDtypeStruct((NW, NPAD), jnp.float32),
    mesh=_mesh,
    compiler_params=_sc_params,
    scratch_types=[
        pltpu.VMEM((EPT,), jnp.int32),
        pltpu.VMEM((NPAD,), jnp.float32),
    ],
)
def _deg_kernel(dst_hbm, part_hbm, dsts, acc):
    c = lax.axis_index("c")
    s = lax.axis_index("s")
    wid = s * NC + c

    @pl.loop(0, NPAD // L)
    def _zero(i):
        acc[pl.ds(i * L, L)] = jnp.zeros((L,), jnp.float32)

    pltpu.sync_copy(dst_hbm.at[pl.ds(wid * EPT, EPT)], dsts)
    ones = jnp.ones((L,), jnp.float32)

    @pl.loop(0, EPT // L)
    def _scatter(j):
        idx = dsts[pl.ds(j * L, L)]
        plsc.addupdate_scatter(acc, [idx], ones)

    pltpu.sync_copy(acc, part_hbm.at[wid])


def _make_scatter(D):
    @functools.partial(
        pl.kernel,
        out_type=jax.ShapeDtypeStruct((NC, NPAD, D), jnp.float32),
        mesh=_mesh,
        compiler_params=_sc_params,
        scratch_types=[
            pltpu.VMEM((K,), jnp.int32),
            pltpu.VMEM((K,), jnp.int32),
            pltpu.VMEM((K, D), jnp.float32),
            pltpu.VMEM_SHARED((NPAD, D), jnp.float32),
            pltpu.SemaphoreType.DMA,
        ],
    )
    def scat(g_hbm, src_hbm, dst_hbm, out_hbm, sidx, didx, rows, accum, sem):
        c = lax.axis_index("c")
        s = lax.axis_index("s")
        wid = s * NC + c

        @pl.loop(0, K)
        def _zero_rows(r):
            for j in range(D // L):
                rows[r, pl.ds(j * L, L)] = jnp.zeros((L,), jnp.float32)

        for t in range(RPT // K):
            pltpu.sync_copy(rows, accum.at[pl.ds(s * RPT + t * K, K)])
        plsc.subcore_barrier()

        @pl.loop(0, NCHUNK)
        def _edges(j):
            base = pl.multiple_of(wid * EPT + j * K, 8)
            pltpu.sync_copy(src_hbm.at[pl.ds(base, K)], sidx)
            pltpu.sync_copy(dst_hbm.at[pl.ds(base, K)], didx)
            pltpu.async_copy(g_hbm.at[sidx], rows, sem).wait()
            pltpu.sync_copy(rows, accum.at[didx], add=True)

        plsc.subcore_barrier()
        for t in range(RPT // K):
            off = s * RPT + t * K
            pltpu.sync_copy(accum.at[pl.ds(off, K)], out_hbm.at[c, pl.ds(off, K)])

    return scat


_scat128 = _make_scatter(128)
_scat64 = _make_scatter(64)


BR = 2048
GRID = NPAD // BR


def _dinv_body(p_ref, o_ref):
    ones = jnp.ones((NW, 1), jnp.float32)
    deg = 1.0 + lax.dot_general(p_ref[...], ones, (((0,), (0,)), ((), ())),
                                preferred_element_type=jnp.float32)
    o_ref[...] = lax.rsqrt(deg)


_dinv_kernel = pl.pallas_call(
    _dinv_body,
    out_shape=jax.ShapeDtypeStruct((NPAD, 1), jnp.float32),
)


def _mm_body(x_ref, w_ref, d_ref, o_ref):
    h = jnp.dot(x_ref[...], w_ref[...], preferred_element_type=jnp.float32,
                precision=lax.Precision.HIGHEST)
    o_ref[...] = d_ref[...] * h


_mm_kernel = pl.pallas_call(
    _mm_body,
    grid=(GRID,),
    in_specs=[
        pl.BlockSpec((BR, 128), lambda i: (i, 0)),
        pl.BlockSpec((128, 128), lambda i: (0, 0)),
        pl.BlockSpec((BR, 1), lambda i: (i, 0)),
    ],
    out_specs=pl.BlockSpec((BR, 128), lambda i: (i, 0)),
    out_shape=jax.ShapeDtypeStruct((NPAD, 128), jnp.float32),
)


def _comb_body(a0_ref, a1_ref, g_ref, d_ref, b_ref, w_ref, o_ref):
    dv = d_ref[...]
    pre = dv * (a0_ref[...] + a1_ref[...] + g_ref[...]) + b_ref[...]
    x2 = jnp.maximum(pre, 0.0)
    h = jnp.dot(x2, w_ref[...], preferred_element_type=jnp.float32,
                precision=lax.Precision.HIGHEST)
    o_ref[...] = dv * h


def _make_combine(Dout):
    return pl.pallas_call(
        _comb_body,
        grid=(GRID,),
        in_specs=[
            pl.BlockSpec((BR, 128), lambda i: (i, 0)),
            pl.BlockSpec((BR, 128), lambda i: (i, 0)),
            pl.BlockSpec((BR, 128), lambda i: (i, 0)),
            pl.BlockSpec((BR, 1), lambda i: (i, 0)),
            pl.BlockSpec((1, 128), lambda i: (0, 0)),
            pl.BlockSpec((128, Dout), lambda i: (0, 0)),
        ],
        out_specs=pl.BlockSpec((BR, Dout), lambda i: (i, 0)),
        out_shape=jax.ShapeDtypeStruct((NPAD, Dout), jnp.float32),
    )


_comb128 = _make_combine(128)
_comb64 = _make_combine(64)


def _final_body(a0_ref, a1_ref, g_ref, d_ref, b_ref, o_ref):
    pre = d_ref[...] * (a0_ref[...] + a1_ref[...] + g_ref[...]) + b_ref[...]
    m = jnp.max(pre, axis=1, keepdims=True)
    e = jnp.exp(pre - m)
    lse = jnp.log(jnp.sum(e, axis=1, keepdims=True)) + m
    o_ref[...] = pre - lse


_final_kernel = pl.pallas_call(
    _final_body,
    grid=(GRID,),
    in_specs=[
        pl.BlockSpec((BR, 64), lambda i: (i, 0)),
        pl.BlockSpec((BR, 64), lambda i: (i, 0)),
        pl.BlockSpec((BR, 64), lambda i: (i, 0)),
        pl.BlockSpec((BR, 1), lambda i: (i, 0)),
        pl.BlockSpec((1, 64), lambda i: (0, 0)),
    ],
    out_specs=pl.BlockSpec((BR, 64), lambda i: (i, 0)),
    out_shape=jax.ShapeDtypeStruct((NPAD, 64), jnp.float32),
)


def kernel(x, edge_index, W1, b1, W2, b2, W3, b3):
    src = edge_index[0]
    dst = edge_index[1]
    x_pad = jnp.zeros((NPAD, 128), jnp.float32).at[:N].set(x)

    part = _deg_kernel(dst)
    dinv = _dinv_kernel(part)

    g1 = _mm_kernel(x_pad, W1, dinv)
    a = _scat128(g1, src, dst)
    g2 = _comb128(a[0], a[1], g1, dinv, b1.reshape(1, 128), W2)
    a = _scat128(g2, src, dst)
    g3 = _comb64(a[0], a[1], g2, dinv, b2.reshape(1, 128), W3)
    a = _scat64(g3, src, dst)
    out = _final_kernel(a[0], a[1], g3, dinv, b3.reshape(1, 64))
    return out[:N]
